# baseline (speedup 1.0000x reference)
import functools

import jax
import jax.numpy as jnp
from jax import lax
from jax.experimental import pallas as pl
from jax.experimental.pallas import tpu as pltpu
from jax.experimental.pallas import tpu_sc as plsc

_N = 4096
_K = 9
_NCH = 4
_E = _N * _K
_EOUT = _E + 256
_EMBED = 128
_NUM_AA = 21
_BIG = 1e10
_ROWS = 256
_ET = _E // 8


def _sub3(u, v):
    return (u[0] - v[0], u[1] - v[1], u[2] - v[2])


def _cross3(u, v):
    return (u[1] * v[2] - u[2] * v[1],
            u[2] * v[0] - u[0] * v[2],
            u[0] * v[1] - u[1] * v[0])


def _dot3(u, v):
    return u[0] * v[0] + u[1] * v[1] + u[2] * v[2]


def _dihedral3(p0, p1, p2, p3, eps=1e-8):
    b1 = _sub3(p1, p0)
    b2 = _sub3(p2, p1)
    b3 = _sub3(p3, p2)
    n1 = _cross3(b1, b2)
    n2 = _cross3(b2, b3)
    b2n = jnp.sqrt(_dot3(b2, b2))
    b2u = tuple(c / (b2n + eps) for c in b2)
    m1 = _cross3(n1, b2u)
    x = _dot3(n1, n2)
    y = _dot3(m1, n2)
    return jnp.arctan2(y, x + eps)


def _node_body(xf_ref, xp_ref, xn_ref, s_ref, rp_ref, emb_ref, inv_ref,
               na_ref, pack_ref):
    i = pl.program_id(0)
    Xf = xf_ref[...]
    Xp = xp_ref[...]
    Xn = xn_ref[...]

    def v3(arr, k):
        return (arr[:, k:k + 1], arr[:, k + 1:k + 2], arr[:, k + 2:k + 3])

    n0 = v3(Xf, 0)
    ca = v3(Xf, 3)
    cc = v3(Xf, 6)
    prev_c = v3(Xp, 6)
    next_n = v3(Xn, 0)
    next_ca = v3(Xn, 3)

    a0 = _dihedral3(prev_c, n0, ca, cc)
    a1 = _dihedral3(n0, ca, cc, next_n)
    a2 = _dihedral3(ca, cc, next_n, next_ca)
    rid = jax.lax.broadcasted_iota(jnp.int32, (_ROWS, 1), 0) + i * _ROWS
    a0 = jnp.where(rid == 0, 0.0, a0)
    a1 = jnp.where(rid == _N - 1, 0.0, a1)
    a2 = jnp.where(rid == _N - 1, 0.0, a2)

    eps = 1e-8
    e1 = _sub3(cc, ca)
    n1 = jnp.sqrt(_dot3(e1, e1))
    e1 = tuple(c / (n1 + eps) for c in e1)
    u = _sub3(n0, ca)
    du = _dot3(u, e1)
    u = (u[0] - du * e1[0], u[1] - du * e1[1], u[2] - du * e1[2])
    nu = jnp.sqrt(_dot3(u, u))
    e2 = tuple(c / (nu + eps) for c in u)
    e3 = _cross3(e1, e2)

    oh = (jax.lax.broadcasted_iota(jnp.int32, (_ROWS, _NUM_AA), 1)
          == s_ref[...]).astype(jnp.float32)
    H = jnp.dot(oh, emb_ref[...], preferred_element_type=jnp.float32)
    arg = rp_ref[...].astype(jnp.float32) * inv_ref[...]
    par = jax.lax.broadcasted_iota(jnp.int32, (_ROWS, _EMBED), 1) % 2 == 0
    H = H + jnp.where(par, jnp.sin(arg), jnp.cos(arg))

    na_ref[:, :_EMBED] = H
    na_ref[:, _EMBED:_EMBED + 1] = a0
    na_ref[:, _EMBED + 1:_EMBED + 2] = a1
    na_ref[:, _EMBED + 2:_EMBED + 3] = a2

    pack_ref[:, 0:12] = Xf
    for k, e in enumerate((e1, e2, e3)):
        for c in range(3):
            pack_ref[:, 12 + 3 * k + c:13 + 3 * k + c] = e[c]
    pack_ref[:, 21:32] = jnp.zeros((_ROWS, 11), jnp.float32)


def _node_features(X, S, RP, emb_table):
    Xf = X.reshape(_N, 12)
    Xp = jnp.roll(Xf, 1, axis=0)
    Xn = jnp.roll(Xf, -1, axis=0)
    j = jnp.arange(_EMBED, dtype=jnp.float32) // 2
    inv = jnp.power(10000.0, -2.0 * j / _EMBED).reshape(1, _EMBED)
    grid = (_N // _ROWS,)
    row = lambda i: (i, 0)
    fix = lambda i: (0, 0)
    return pl.pallas_call(
        _node_body,
        grid=grid,
        in_specs=[
            pl.BlockSpec((_ROWS, 12), row),
            pl.BlockSpec((_ROWS, 12), row),
            pl.BlockSpec((_ROWS, 12), row),
            pl.BlockSpec((_ROWS, 1), row),
            pl.BlockSpec((_ROWS, 1), row),
            pl.BlockSpec((_NUM_AA, _EMBED), fix),
            pl.BlockSpec((1, _EMBED), fix),
        ],
        out_specs=[
            pl.BlockSpec((_ROWS, _EMBED + 3), row),
            pl.BlockSpec((_ROWS, 32), row),
        ],
        out_shape=[
            jax.ShapeDtypeStruct((_N, _EMBED + 3), jnp.float32),
            jax.ShapeDtypeStruct((_N, 32), jnp.float32),
        ],
    )(Xf, Xp, Xn, S.reshape(_N, 1), RP.reshape(_N, 1), emb_table, inv)


_CH = 512


def _dist_topk_body(xr_ref, xt_ref, bidr_ref, segr_ref, bidc_ref, segc_ref,
                    vals_ref, idx_ref, cnt_ref, dist_ref, macc_ref, iacc_ref):
    i = pl.program_id(0)
    Xi = xr_ref[...]
    XT = xt_ref[...]
    nb = jnp.sum(XT * XT, axis=0, keepdims=True)

    colid = jax.lax.broadcasted_iota(jnp.int32, (_ROWS, _N), 1)
    rowid1 = jax.lax.broadcasted_iota(jnp.int32, (_ROWS, 1), 0) + i * _ROWS
    match = (bidr_ref[...] == bidc_ref[...]) & (segr_ref[...] == segc_ref[...])
    lo = jnp.min(jnp.where(match, colid, _N), axis=1, keepdims=True)
    hi = jnp.max(jnp.where(match, colid, -1), axis=1, keepdims=True) + 1
    tile_lo = jnp.min(lo)
    tile_hi = jnp.max(hi)
    cntv = hi - lo - 1

    dist_ref[...] = jnp.full((_ROWS, _N), _BIG, jnp.float32)
    nchunks = _N // _CH
    for c in range(nchunks):
        c0 = c * _CH

        @pl.when((tile_lo < c0 + _CH) & (tile_hi > c0))
        def _(c0=c0):
            d2 = jnp.full((_ROWS, _CH), jnp.inf, dtype=jnp.float32)
            for a in range(_NCH):
                Xa = Xi[:, 3 * a:3 * a + 3]
                na = jnp.sum(Xa * Xa, axis=1, keepdims=True)
                for b in range(_NCH):
                    XTb = XT[:, b * _N + c0:b * _N + c0 + _CH]
                    P = jnp.dot(Xa, XTb, preferred_element_type=jnp.float32)
                    cur = na + nb[:, b * _N + c0:b * _N + c0 + _CH] - 2.0 * P
                    d2 = jnp.minimum(d2, cur)
            dc = jnp.sqrt(jnp.maximum(d2, 0.0))
            cid = (jax.lax.broadcasted_iota(jnp.int32, (_ROWS, _CH), 1) + c0)
            dc = jnp.where(match[:, c0:c0 + _CH], dc, _BIG)
            dc = jnp.where(cid == rowid1, dc + _BIG, dc)
            dist_ref[:, c0:c0 + _CH] = dc

    cnt_ref[...] = jnp.minimum(cntv, _K)
    for r in range(_K):
        macc_ref[...] = jnp.full((_ROWS, 1), _BIG, jnp.float32)
        for c in range(nchunks):
            c0 = c * _CH

            @pl.when((tile_lo < c0 + _CH) & (tile_hi > c0))
            def _(c0=c0):
                cmin = jnp.min(dist_ref[:, c0:c0 + _CH], axis=1, keepdims=True)
                macc_ref[...] = jnp.minimum(macc_ref[...], cmin)
        m = macc_ref[...]
        iacc_ref[...] = jnp.full((_ROWS, 1), _N, jnp.int32)
        for c in range(nchunks):
            c0 = c * _CH

            @pl.when((tile_lo < c0 + _CH) & (tile_hi > c0))
            def _(c0=c0):
                dc = dist_ref[:, c0:c0 + _CH]
                cid = (jax.lax.broadcasted_iota(jnp.int32, (_ROWS, _CH), 1)
                       + c0)
                carg = jnp.min(jnp.where(dc == m, cid, _N), axis=1,
                               keepdims=True)
                iacc_ref[...] = jnp.minimum(iacc_ref[...], carg)
        idxp = iacc_ref[...]
        for c in range(nchunks):
            c0 = c * _CH

            @pl.when((tile_lo < c0 + _CH) & (tile_hi > c0))
            def _(c0=c0):
                cid = (jax.lax.broadcasted_iota(jnp.int32, (_ROWS, _CH), 1)
                       + c0)
                dist_ref[:, c0:c0 + _CH] = jnp.where(
                    cid == idxp, jnp.inf, dist_ref[:, c0:c0 + _CH])
        t = r - cntv
        closed = jnp.where(t < lo, t, jnp.where(t == lo, rowid1,
                                                hi + t - lo - 1))
        finite = cntv > r
        vals_ref[:, r:r + 1] = jnp.where(finite, m, _BIG)
        idx_ref[:, r:r + 1] = jnp.where(finite, idxp, closed)


def _dist_topk(X, bid, Seg):
    Xf = X.reshape(_N, 12)
    XT = jnp.transpose(X, (2, 1, 0)).reshape(3, _NCH * _N)
    row = lambda i: (i, 0)
    fix = lambda i: (0, 0)
    return pl.pallas_call(
        _dist_topk_body,
        grid=(_N // _ROWS,),
        in_specs=[
            pl.BlockSpec((_ROWS, 12), row),
            pl.BlockSpec((3, _NCH * _N), fix),
            pl.BlockSpec((_ROWS, 1), row),
            pl.BlockSpec((_ROWS, 1), row),
            pl.BlockSpec((1, _N), fix),
            pl.BlockSpec((1, _N), fix),
        ],
        out_specs=[
            pl.BlockSpec((_ROWS, _K), row),
            pl.BlockSpec((_ROWS, _K), row),
            pl.BlockSpec((_ROWS, 1), row),
        ],
        out_shape=[
            jax.ShapeDtypeStruct((_N, _K), jnp.float32),
            jax.ShapeDtypeStruct((_N, _K), jnp.int32),
            jax.ShapeDtypeStruct((_N, 1), jnp.int32),
        ],
        scratch_shapes=[
            pltpu.VMEM((_ROWS, _N), jnp.float32),
            pltpu.VMEM((_ROWS, 1), jnp.float32),
            pltpu.VMEM((_ROWS, 1), jnp.int32),
        ],
    )(Xf, XT, bid.reshape(_N, 1), Seg.reshape(_N, 1),
      bid.reshape(1, _N), Seg.reshape(1, _N))


def _pos_body(cntc_ref, cntr_ref, idx_ref, pos_ref, ipay_ref, spay_ref,
              tot_ref):
    i = pl.program_id(0)
    cnt = cntc_ref[...]
    cntrow = cntr_ref[...]
    colid = jax.lax.broadcasted_iota(jnp.int32, (_ROWS, _N), 1)
    rid2 = jax.lax.broadcasted_iota(jnp.int32, (_ROWS, _N), 0) + i * _ROWS
    off = jnp.sum(jnp.where(colid < rid2, cntrow, 0), axis=1, keepdims=True)
    rid = jax.lax.broadcasted_iota(jnp.int32, (_ROWS, 1), 0) + i * _ROWS
    zero14 = jnp.zeros((_ROWS, 14), jnp.int32)
    for r in range(_K):
        valid_r = cnt > r
        pos_ref[:, r:r + 1] = jnp.where(valid_r, off + r, _E + r)
        ipay_ref[:, 16 * r:16 * r + 1] = idx_ref[:, r:r + 1]
        ipay_ref[:, 16 * r + 1:16 * r + 2] = rid
        ipay_ref[:, 16 * r + 2:16 * (r + 1)] = zero14
        spay_ref[:, 16 * r:16 * r + 1] = rid * _K + r
        spay_ref[:, 16 * r + 1:16 * (r + 1)] = jnp.zeros((_ROWS, 15), jnp.int32)
    tot = off[_ROWS - 1:_ROWS, :] + cnt[_ROWS - 1:_ROWS, :]
    tot_ref[...] = jnp.broadcast_to(tot, (1, 16))


def _positions(cnt, idxs):
    row = lambda i: (i, 0)
    fix = lambda i: (0, 0)
    return pl.pallas_call(
        _pos_body,
        grid=(_N // _ROWS,),
        in_specs=[
            pl.BlockSpec((_ROWS, 1), row),
            pl.BlockSpec((1, _N), fix),
            pl.BlockSpec((_ROWS, _K), row),
        ],
        out_specs=[
            pl.BlockSpec((_ROWS, _K), row),
            pl.BlockSpec((_ROWS, 16 * _K), row),
            pl.BlockSpec((_ROWS, 16 * _K), row),
            pl.BlockSpec((1, 16), fix),
        ],
        out_shape=[
            jax.ShapeDtypeStruct((_N, _K), jnp.int32),
            jax.ShapeDtypeStruct((_N, 16 * _K), jnp.int32),
            jax.ShapeDtypeStruct((_N, 16 * _K), jnp.int32),
            jax.ShapeDtypeStruct((1, 16), jnp.int32),
        ],
    )(cnt, cnt.reshape(1, _N), idxs)


def _edge_geom_body(gd_ref, gs_ref, vals_ref, attr_ref, v_ref):
    gd = gd_ref[...]
    gs = gs_ref[...]

    def r3(arr, k):
        return (arr[k:k + 1, :], arr[k + 1:k + 2, :], arr[k + 2:k + 3, :])

    xd_n, xd_ca, xd_c = r3(gd, 0), r3(gd, 3), r3(gd, 6)
    xs_n, xs_ca, xs_c = r3(gs, 0), r3(gs, 3), r3(gs, 6)
    phi = _dihedral3(xs_c, xd_n, xd_ca, xd_c)
    psi = _dihedral3(xs_n, xs_ca, xs_c, xd_n)
    attr_ref[0:1, :] = phi
    attr_ref[1:2, :] = psi

    ed = [r3(gd, 12 + 3 * k) for k in range(3)]
    es = [r3(gs, 12 + 3 * k) for k in range(3)]
    u = _sub3(xs_ca, xd_ca)
    for i in range(3):
        attr_ref[2 + i:3 + i, :] = _dot3(ed[i], u)
    for i in range(3):
        for k in range(3):
            attr_ref[5 + 3 * i + k:6 + 3 * i + k, :] = _dot3(ed[i], es[k])

    diff = _sub3(xd_ca, xs_ca)
    dsq = _dot3(diff, diff)
    d_rad = jnp.sqrt(dsq + 1e-8)
    for s in range(15):
        attr_ref[14 + s:15 + s, :] = jnp.exp(-(d_rad - float(s)) ** 2)
    attr_ref[29:32, :] = jnp.zeros((3, _ET), jnp.float32)

    dn = jnp.sqrt(dsq) + 1e-8
    valid = (vals_ref[...] < _BIG).astype(jnp.float32)
    inv2 = 1.0 / (dn * dn)
    inv3 = inv2 / dn
    inv4 = inv2 * inv2
    for o, invo in enumerate((inv2, inv3, inv4)):
        for c in range(3):
            v_ref[3 * o + c:3 * o + c + 1, :] = diff[c] * invo * valid
    v_ref[9:16, :] = jnp.zeros((7, _ET), jnp.float32)


def _edge_geom(gdT, gsT, vals_row):
    col = lambda i: (0, i)
    return pl.pallas_call(
        _edge_geom_body,
        grid=(_E // _ET,),
        in_specs=[
            pl.BlockSpec((32, _ET), col),
            pl.BlockSpec((32, _ET), col),
            pl.BlockSpec((1, _ET), col),
        ],
        out_specs=[
            pl.BlockSpec((32, _ET), col),
            pl.BlockSpec((16, _ET), col),
        ],
        out_shape=[
            jax.ShapeDtypeStruct((32, _E), jnp.float32),
            jax.ShapeDtypeStruct((16, _E), jnp.float32),
        ],
    )(gdT, gsT, vals_row)


def _nforce_body(p0_ref, p1_ref, tot_ref, g0_ref, k0_ref, d00_ref, nv_ref):
    agg = p0_ref[...] + p1_ref[...]
    npad = (_E - tot_ref[0, 0]).astype(jnp.float32)
    g0 = g0_ref[...]
    k0 = k0_ref[...]
    diff0 = [g0[:, 3 + c:4 + c] - k0[:, 3 + c:4 + c] for c in range(3)]
    dsq0 = diff0[0] * diff0[0] + diff0[1] * diff0[1] + diff0[2] * diff0[2]
    dn0 = jnp.sqrt(dsq0) + 1e-8
    i2 = 1.0 / (dn0 * dn0)
    i3 = i2 / dn0
    i4 = i2 * i2
    rowmask = (jax.lax.broadcasted_iota(jnp.int32, (_N, 1), 0) == d00_ref[0, 0])
    cols = []
    for o, invo in enumerate((i2, i3, i4)):
        ac = [agg[:, 3 * o + c:3 * o + c + 1]
              + jnp.where(rowmask, npad * diff0[c] * invo, 0.0)
              for c in range(3)]
        nrm = jnp.sqrt(ac[0] * ac[0] + ac[1] * ac[1] + ac[2] * ac[2]) + 1e-8
        cols.extend([a / nrm for a in ac])
    for c, col in enumerate(cols):
        nv_ref[:, c:c + 1] = col
    nv_ref[:, 9:16] = jnp.zeros((_N, 7), jnp.float32)


def _nforce_finish(part0, part1, tot16, gath0, pack0, dst00):
    return pl.pallas_call(
        _nforce_body,
        out_shape=jax.ShapeDtypeStruct((_N, 16), jnp.float32),
    )(part0, part1, tot16, gath0, pack0, dst00)


def _assemble_body(attr_ref, nd_ref, ns_ref, pay_ref):
    pay_ref[0:29, :] = attr_ref[0:29, :]
    nd = nd_ref[...]
    ns = ns_ref[...]
    for o in range(3):
        prod = (nd[3 * o:3 * o + 1, :] * ns[3 * o:3 * o + 1, :]
                + nd[3 * o + 1:3 * o + 2, :] * ns[3 * o + 1:3 * o + 2, :]
                + nd[3 * o + 2:3 * o + 3, :] * ns[3 * o + 2:3 * o + 3, :])
        pay_ref[29 + o:30 + o, :] = prod


def _assemble(attrT, ndT, nsT):
    col = lambda i: (0, i)
    return pl.pallas_call(
        _assemble_body,
        grid=(_E // _ET,),
        in_specs=[
            pl.BlockSpec((32, _ET), col),
            pl.BlockSpec((16, _ET), col),
            pl.BlockSpec((16, _ET), col),
        ],
        out_specs=pl.BlockSpec((32, _ET), col),
        out_shape=jax.ShapeDtypeStruct((32, _E), jnp.float32),
    )(attrT, ndT, nsT)


_NW = 32
_BW = _E // _NW
_SC_MESH = dict(core_axis_name="c", subcore_axis_name="s")
_SC_PARAMS = pltpu.CompilerParams(use_tc_tiling_on_sc=False)


def _wid():
    return lax.axis_index("s") * 2 + lax.axis_index("c")


def _sc_gather(table, idx):
    D = table.shape[1]

    @functools.partial(
        pl.kernel,
        out_type=jax.ShapeDtypeStruct((_E, D), table.dtype),
        mesh=plsc.VectorSubcoreMesh(**_SC_MESH),
        compiler_params=_SC_PARAMS,
        scratch_types=[
            pltpu.VMEM((_BW,), jnp.int32),
            pltpu.VMEM((_BW, D), table.dtype),
            pltpu.SemaphoreType.DMA,
        ],
    )
    def k(table_hbm, idx_hbm, out_hbm, idx_v, rows_v, sem):
        base = _wid() * _BW
        pltpu.sync_copy(idx_hbm.at[pl.ds(base, _BW)], idx_v)
        pltpu.async_copy(table_hbm.at[idx_v], rows_v, sem).wait()
        pltpu.sync_copy(rows_v, out_hbm.at[pl.ds(base, _BW)])

    return k(table, idx)


def _sc_scatter_add(v, dst):
    zeros = jnp.zeros((_N, 16), jnp.float32)
    rows_per_s = _N // 16

    @functools.partial(
        pl.kernel,
        out_type=jax.ShapeDtypeStruct((2 * _N, 16), jnp.float32),
        mesh=plsc.VectorSubcoreMesh(**_SC_MESH),
        compiler_params=_SC_PARAMS,
        scratch_types=[
            pltpu.VMEM_SHARED((_N, 16), jnp.float32),
            pltpu.VMEM((_BW, 16), jnp.float32),
            pltpu.VMEM((_BW,), jnp.int32),
        ],
    )
    def k(v_hbm, dst_hbm, z_hbm, out_hbm, shared, v_v, idx_v):
        cid = lax.axis_index("c")
        sid = lax.axis_index("s")
        srow = sid * rows_per_s
        pltpu.sync_copy(z_hbm.at[pl.ds(srow, rows_per_s)],
                        shared.at[pl.ds(srow, rows_per_s)])
        plsc.subcore_barrier()
        base = _wid() * _BW
        pltpu.sync_copy(dst_hbm.at[pl.ds(base, _BW)], idx_v)
        pltpu.sync_copy(v_hbm.at[pl.ds(base, _BW)], v_v)
        pltpu.sync_copy(v_v, shared.at[idx_v], add=True)
        plsc.subcore_barrier()
        pltpu.sync_copy(shared.at[pl.ds(srow, rows_per_s)],
                        out_hbm.at[pl.ds(cid * _N + srow, rows_per_s)])

    out = k(v, dst, zeros)
    return out[:_N], out[_N:]


def _sc_scatter_slots(spay2d, pos_flat):
    @functools.partial(
        pl.kernel,
        out_type=jax.ShapeDtypeStruct((_EOUT, 16), jnp.int32),
        mesh=plsc.VectorSubcoreMesh(**_SC_MESH),
        compiler_params=_SC_PARAMS,
        scratch_types=[
            pltpu.VMEM((_BW,), jnp.int32),
            pltpu.VMEM((_BW, 16), jnp.int32),
        ],
    )
    def k(pay_hbm, pos_hbm, out_hbm, pos_v, rows_v):
        base = _wid() * _BW
        pltpu.sync_copy(pos_hbm.at[pl.ds(base, _BW)], pos_v)
        pltpu.sync_copy(pay_hbm.at[pl.ds(base, _BW)], rows_v)
        pltpu.sync_copy(rows_v, out_hbm.at[pos_v])

    return k(spay2d, pos_flat)


def _sel_body(scat_ref, tot_ref, sel_ref):
    i = pl.program_id(0)
    rows = scat_ref.shape[0]
    p = jax.lax.broadcasted_iota(jnp.int32, (rows, 1), 0) + i * rows
    sel_ref[...] = jnp.where(p < tot_ref[0, 0], scat_ref[:, 0:1], 0)


def _sel_from_scat(scat, tot16):
    rows = _E // 16
    return pl.pallas_call(
        _sel_body,
        grid=(16,),
        in_specs=[
            pl.BlockSpec((rows, 16), lambda i: (i, 0)),
            pl.BlockSpec((1, 16), lambda i: (0, 0)),
        ],
        out_specs=pl.BlockSpec((rows, 1), lambda i: (i, 0)),
        out_shape=jax.ShapeDtypeStruct((_E, 1), jnp.int32),
    )(scat[:_E], tot16)


def kernel(X, S, RP, Seg, bid, emb_table):
    node_attr, pack = _node_features(X, S, RP, emb_table)
    vals, idxs, cnt = _dist_topk(X, bid, Seg)
    pos, ipay, spay, tot16 = _positions(cnt, idxs)

    dst_flat = idxs.reshape(_E)
    gath = _sc_gather(pack, dst_flat)

    gdT = gath.T
    gsT = jnp.broadcast_to(pack.T[:, :, None], (32, _N, _K)).reshape(32, _E)
    vals_row = vals.reshape(1, _E)
    attrT, vT = _edge_geom(gdT, gsT, vals_row)

    part0, part1 = _sc_scatter_add(vT.T, dst_flat)
    nvecs = _nforce_finish(part0, part1, tot16,
                           gath[0:1, :], pack[0:1, :], idxs[0:1, 0:1])

    nvd = _sc_gather(nvecs, dst_flat)
    nvdT = nvd.T
    nvsT = jnp.broadcast_to(nvecs.T[:, :, None], (16, _N, _K)).reshape(16, _E)
    payT = _assemble(attrT, nvdT, nvsT)

    gpay = payT.T
    ipay2d = ipay.reshape(_E, 16)
    spay2d = spay.reshape(_E, 16)
    scat = _sc_scatter_slots(spay2d, pos.reshape(_E))
    sel = _sel_from_scat(scat, tot16).reshape(_E)
    edge_attr = _sc_gather(gpay, sel)
    ints = _sc_gather(ipay2d, sel)
    edges = jnp.stack([ints[:, 0], ints[:, 1]], axis=0)
    return (node_attr, edges, edge_attr)


# speedup vs baseline: 1.3506x; 1.3506x over previous
import functools

import jax
import jax.numpy as jnp
from jax import lax
from jax.experimental import pallas as pl
from jax.experimental.pallas import tpu as pltpu
from jax.experimental.pallas import tpu_sc as plsc

_N = 4096
_K = 9
_NCH = 4
_E = _N * _K
_EOUT = _E + 256
_EMBED = 128
_NUM_AA = 21
_BIG = 1e10
_ROWS = 256
_ET = _E // 8


def _sub3(u, v):
    return (u[0] - v[0], u[1] - v[1], u[2] - v[2])


def _cross3(u, v):
    return (u[1] * v[2] - u[2] * v[1],
            u[2] * v[0] - u[0] * v[2],
            u[0] * v[1] - u[1] * v[0])


def _dot3(u, v):
    return u[0] * v[0] + u[1] * v[1] + u[2] * v[2]


def _dihedral3(p0, p1, p2, p3, eps=1e-8):
    b1 = _sub3(p1, p0)
    b2 = _sub3(p2, p1)
    b3 = _sub3(p3, p2)
    n1 = _cross3(b1, b2)
    n2 = _cross3(b2, b3)
    b2n = jnp.sqrt(_dot3(b2, b2))
    b2u = tuple(c / (b2n + eps) for c in b2)
    m1 = _cross3(n1, b2u)
    x = _dot3(n1, n2)
    y = _dot3(m1, n2)
    return jnp.arctan2(y, x + eps)


def _hsin_body(s_ref, rp_ref, emb_ref, inv_ref, h_ref):
    oh = (jax.lax.broadcasted_iota(jnp.int32, (_ROWS, _NUM_AA), 1)
          == s_ref[...]).astype(jnp.float32)
    H = jnp.dot(oh, emb_ref[...], preferred_element_type=jnp.float32)
    arg = rp_ref[...].astype(jnp.float32) * inv_ref[...]
    par = jax.lax.broadcasted_iota(jnp.int32, (_ROWS, _EMBED), 1) % 2 == 0
    h_ref[...] = H + jnp.where(par, jnp.sin(arg), jnp.cos(arg))


def _hsin(S, RP, emb_table):
    j = jnp.arange(_EMBED, dtype=jnp.float32) // 2
    inv = jnp.power(10000.0, -2.0 * j / _EMBED).reshape(1, _EMBED)
    row = lambda i: (i, 0)
    fix = lambda i: (0, 0)
    return pl.pallas_call(
        _hsin_body,
        grid=(_N // _ROWS,),
        in_specs=[
            pl.BlockSpec((_ROWS, 1), row),
            pl.BlockSpec((_ROWS, 1), row),
            pl.BlockSpec((_NUM_AA, _EMBED), fix),
            pl.BlockSpec((1, _EMBED), fix),
        ],
        out_specs=pl.BlockSpec((_ROWS, _EMBED), row),
        out_shape=jax.ShapeDtypeStruct((_N, _EMBED), jnp.float32),
    )(S.reshape(_N, 1), RP.reshape(_N, 1), emb_table, inv)


def _geo_body(xf_ref, xp_ref, xn_ref, ang_ref, frm_ref):
    xf = xf_ref[...]
    xp = xp_ref[...]
    xn = xn_ref[...]

    def r3(arr, k):
        return (arr[k:k + 1, :], arr[k + 1:k + 2, :], arr[k + 2:k + 3, :])

    n0, ca, cc = r3(xf, 0), r3(xf, 3), r3(xf, 6)
    prev_c = r3(xp, 6)
    next_n, next_ca = r3(xn, 0), r3(xn, 3)

    a0 = _dihedral3(prev_c, n0, ca, cc)
    a1 = _dihedral3(n0, ca, cc, next_n)
    a2 = _dihedral3(ca, cc, next_n, next_ca)
    rid = jax.lax.broadcasted_iota(jnp.int32, (1, _N), 1)
    ang_ref[0:1, :] = jnp.where(rid == 0, 0.0, a0)
    ang_ref[1:2, :] = jnp.where(rid == _N - 1, 0.0, a1)
    ang_ref[2:3, :] = jnp.where(rid == _N - 1, 0.0, a2)

    eps = 1e-8
    e1 = _sub3(cc, ca)
    n1 = jnp.sqrt(_dot3(e1, e1))
    e1 = tuple(c / (n1 + eps) for c in e1)
    u = _sub3(n0, ca)
    du = _dot3(u, e1)
    u = (u[0] - du * e1[0], u[1] - du * e1[1], u[2] - du * e1[2])
    nu = jnp.sqrt(_dot3(u, u))
    e2 = tuple(c / (nu + eps) for c in u)
    e3 = _cross3(e1, e2)
    for k, e in enumerate((e1, e2, e3)):
        for c in range(3):
            frm_ref[3 * k + c:3 * k + c + 1, :] = e[c]


def _geometry(XfT):
    XpT = jnp.roll(XfT, 1, axis=1)
    XnT = jnp.roll(XfT, -1, axis=1)
    return pl.pallas_call(
        _geo_body,
        out_shape=[
            jax.ShapeDtypeStruct((3, _N), jnp.float32),
            jax.ShapeDtypeStruct((9, _N), jnp.float32),
        ],
    )(XfT, XpT, XnT)


def _dist_topk_body(xr_ref, xt_ref, bidr_ref, segr_ref, bidc_ref, segc_ref,
                    vals_ref, idx_ref, cnt_ref):
    i = pl.program_id(0)
    Xi = xr_ref[...]
    XT = xt_ref[...]
    nb = jnp.sum(XT * XT, axis=0, keepdims=True)
    d2 = jnp.full((_ROWS, _N), jnp.inf, dtype=jnp.float32)
    for a in range(_NCH):
        Xa = Xi[:, 3 * a:3 * a + 3]
        na = jnp.sum(Xa * Xa, axis=1, keepdims=True)
        for b in range(_NCH):
            XTb = XT[:, b * _N:(b + 1) * _N]
            P = jnp.dot(Xa, XTb, preferred_element_type=jnp.float32)
            cur = na + nb[:, b * _N:(b + 1) * _N] - 2.0 * P
            d2 = jnp.minimum(d2, cur)
    dist = jnp.sqrt(jnp.maximum(d2, 0.0))
    mask = (bidr_ref[...] == bidc_ref[...]) & (segr_ref[...] == segc_ref[...])
    dist = jnp.where(mask, dist, _BIG)
    colid = jax.lax.broadcasted_iota(jnp.int32, (_ROWS, _N), 1)
    rowid = jax.lax.broadcasted_iota(jnp.int32, (_ROWS, _N), 0) + i * _ROWS
    dist = jnp.where(colid == rowid, dist + _BIG, dist)
    cnt = jnp.zeros((_ROWS, 1), jnp.int32)
    for r in range(_K):
        m = jnp.min(dist, axis=1, keepdims=True)
        idx = jnp.min(jnp.where(dist == m, colid, _N), axis=1, keepdims=True)
        vals_ref[:, r:r + 1] = m
        idx_ref[:, r:r + 1] = idx
        cnt += (m < _BIG).astype(jnp.int32)
        dist = jnp.where(colid == idx, jnp.inf, dist)
    cnt_ref[...] = cnt


def _dist_topk(X, bid, Seg):
    Xf = X.reshape(_N, 12)
    XT = jnp.transpose(X, (2, 1, 0)).reshape(3, _NCH * _N)
    row = lambda i: (i, 0)
    fix = lambda i: (0, 0)
    return pl.pallas_call(
        _dist_topk_body,
        grid=(_N // _ROWS,),
        in_specs=[
            pl.BlockSpec((_ROWS, 12), row),
            pl.BlockSpec((3, _NCH * _N), fix),
            pl.BlockSpec((_ROWS, 1), row),
            pl.BlockSpec((_ROWS, 1), row),
            pl.BlockSpec((1, _N), fix),
            pl.BlockSpec((1, _N), fix),
        ],
        out_specs=[
            pl.BlockSpec((_ROWS, _K), row),
            pl.BlockSpec((_ROWS, _K), row),
            pl.BlockSpec((_ROWS, 1), row),
        ],
        out_shape=[
            jax.ShapeDtypeStruct((_N, _K), jnp.float32),
            jax.ShapeDtypeStruct((_N, _K), jnp.int32),
            jax.ShapeDtypeStruct((_N, 1), jnp.int32),
        ],
    )(Xf, XT, bid.reshape(_N, 1), Seg.reshape(_N, 1),
      bid.reshape(1, _N), Seg.reshape(1, _N))


def _pos_body(cntc_ref, cntr_ref, idx_ref, pos_ref, ipay_ref, spay_ref,
              tot_ref):
    i = pl.program_id(0)
    cnt = cntc_ref[...]
    cntrow = cntr_ref[...]
    colid = jax.lax.broadcasted_iota(jnp.int32, (_ROWS, _N), 1)
    rid2 = jax.lax.broadcasted_iota(jnp.int32, (_ROWS, _N), 0) + i * _ROWS
    off = jnp.sum(jnp.where(colid < rid2, cntrow, 0), axis=1, keepdims=True)
    rid = jax.lax.broadcasted_iota(jnp.int32, (_ROWS, 1), 0) + i * _ROWS
    zero14 = jnp.zeros((_ROWS, 14), jnp.int32)
    for r in range(_K):
        valid_r = cnt > r
        pos_ref[:, r:r + 1] = jnp.where(valid_r, off + r, _E + r)
        ipay_ref[:, 16 * r:16 * r + 1] = idx_ref[:, r:r + 1]
        ipay_ref[:, 16 * r + 1:16 * r + 2] = rid
        ipay_ref[:, 16 * r + 2:16 * (r + 1)] = zero14
        spay_ref[:, 16 * r:16 * r + 1] = rid * _K + r
        spay_ref[:, 16 * r + 1:16 * (r + 1)] = jnp.zeros((_ROWS, 15), jnp.int32)
    tot = off[_ROWS - 1:_ROWS, :] + cnt[_ROWS - 1:_ROWS, :]
    tot_ref[...] = jnp.broadcast_to(tot, (1, 16))


def _positions(cnt, idxs):
    row = lambda i: (i, 0)
    fix = lambda i: (0, 0)
    return pl.pallas_call(
        _pos_body,
        grid=(_N // _ROWS,),
        in_specs=[
            pl.BlockSpec((_ROWS, 1), row),
            pl.BlockSpec((1, _N), fix),
            pl.BlockSpec((_ROWS, _K), row),
        ],
        out_specs=[
            pl.BlockSpec((_ROWS, _K), row),
            pl.BlockSpec((_ROWS, 16 * _K), row),
            pl.BlockSpec((_ROWS, 16 * _K), row),
            pl.BlockSpec((1, 16), fix),
        ],
        out_shape=[
            jax.ShapeDtypeStruct((_N, _K), jnp.int32),
            jax.ShapeDtypeStruct((_N, 16 * _K), jnp.int32),
            jax.ShapeDtypeStruct((_N, 16 * _K), jnp.int32),
            jax.ShapeDtypeStruct((1, 16), jnp.int32),
        ],
    )(cnt, cnt.reshape(1, _N), idxs)


def _edge_geom_body(gd_ref, gs_ref, vals_ref, attr_ref, v_ref):
    gd = gd_ref[...]
    gs = gs_ref[...]

    def r3(arr, k):
        return (arr[k:k + 1, :], arr[k + 1:k + 2, :], arr[k + 2:k + 3, :])

    xd_n, xd_ca, xd_c = r3(gd, 0), r3(gd, 3), r3(gd, 6)
    xs_n, xs_ca, xs_c = r3(gs, 0), r3(gs, 3), r3(gs, 6)
    phi = _dihedral3(xs_c, xd_n, xd_ca, xd_c)
    psi = _dihedral3(xs_n, xs_ca, xs_c, xd_n)
    attr_ref[0:1, :] = phi
    attr_ref[1:2, :] = psi

    ed = [r3(gd, 12 + 3 * k) for k in range(3)]
    es = [r3(gs, 12 + 3 * k) for k in range(3)]
    u = _sub3(xs_ca, xd_ca)
    for i in range(3):
        attr_ref[2 + i:3 + i, :] = _dot3(ed[i], u)
    for i in range(3):
        for k in range(3):
            attr_ref[5 + 3 * i + k:6 + 3 * i + k, :] = _dot3(ed[i], es[k])

    diff = _sub3(xd_ca, xs_ca)
    dsq = _dot3(diff, diff)
    d_rad = jnp.sqrt(dsq + 1e-8)
    for s in range(15):
        attr_ref[14 + s:15 + s, :] = jnp.exp(-(d_rad - float(s)) ** 2)
    attr_ref[29:32, :] = jnp.zeros((3, _ET), jnp.float32)

    dn = jnp.sqrt(dsq) + 1e-8
    valid = (vals_ref[...] < _BIG).astype(jnp.float32)
    inv2 = 1.0 / (dn * dn)
    inv3 = inv2 / dn
    inv4 = inv2 * inv2
    for o, invo in enumerate((inv2, inv3, inv4)):
        for c in range(3):
            v_ref[3 * o + c:3 * o + c + 1, :] = diff[c] * invo * valid
    v_ref[9:16, :] = jnp.zeros((7, _ET), jnp.float32)


def _edge_geom(gdT, gsT, vals_row):
    col = lambda i: (0, i)
    return pl.pallas_call(
        _edge_geom_body,
        grid=(_E // _ET,),
        in_specs=[
            pl.BlockSpec((32, _ET), col),
            pl.BlockSpec((32, _ET), col),
            pl.BlockSpec((1, _ET), col),
        ],
        out_specs=[
            pl.BlockSpec((32, _ET), col),
            pl.BlockSpec((16, _ET), col),
        ],
        out_shape=[
            jax.ShapeDtypeStruct((32, _E), jnp.float32),
            jax.ShapeDtypeStruct((16, _E), jnp.float32),
        ],
    )(gdT, gsT, vals_row)


def _nforce_body(p0_ref, p1_ref, tot_ref, g0_ref, k0_ref, d00_ref, nv_ref):
    agg = p0_ref[...] + p1_ref[...]
    npad = (_E - tot_ref[0, 0]).astype(jnp.float32)
    g0 = g0_ref[...]
    k0 = k0_ref[...]
    diff0 = [g0[:, 3 + c:4 + c] - k0[:, 3 + c:4 + c] for c in range(3)]
    dsq0 = diff0[0] * diff0[0] + diff0[1] * diff0[1] + diff0[2] * diff0[2]
    dn0 = jnp.sqrt(dsq0) + 1e-8
    i2 = 1.0 / (dn0 * dn0)
    i3 = i2 / dn0
    i4 = i2 * i2
    rowmask = (jax.lax.broadcasted_iota(jnp.int32, (_N, 1), 0) == d00_ref[0, 0])
    cols = []
    for o, invo in enumerate((i2, i3, i4)):
        ac = [agg[:, 3 * o + c:3 * o + c + 1]
              + jnp.where(rowmask, npad * diff0[c] * invo, 0.0)
              for c in range(3)]
        nrm = jnp.sqrt(ac[0] * ac[0] + ac[1] * ac[1] + ac[2] * ac[2]) + 1e-8
        cols.extend([a / nrm for a in ac])
    for c, col in enumerate(cols):
        nv_ref[:, c:c + 1] = col
    nv_ref[:, 9:16] = jnp.zeros((_N, 7), jnp.float32)


def _nforce_finish(part0, part1, tot16, gath0, pack0, dst00):
    return pl.pallas_call(
        _nforce_body,
        out_shape=jax.ShapeDtypeStruct((_N, 16), jnp.float32),
    )(part0, part1, tot16, gath0, pack0, dst00)


def _assemble_body(attr_ref, nd_ref, ns_ref, pay_ref):
    pay_ref[0:29, :] = attr_ref[0:29, :]
    nd = nd_ref[...]
    ns = ns_ref[...]
    for o in range(3):
        prod = (nd[3 * o:3 * o + 1, :] * ns[3 * o:3 * o + 1, :]
                + nd[3 * o + 1:3 * o + 2, :] * ns[3 * o + 1:3 * o + 2, :]
                + nd[3 * o + 2:3 * o + 3, :] * ns[3 * o + 2:3 * o + 3, :])
        pay_ref[29 + o:30 + o, :] = prod


def _assemble(attrT, ndT, nsT):
    col = lambda i: (0, i)
    return pl.pallas_call(
        _assemble_body,
        grid=(_E // _ET,),
        in_specs=[
            pl.BlockSpec((32, _ET), col),
            pl.BlockSpec((16, _ET), col),
            pl.BlockSpec((16, _ET), col),
        ],
        out_specs=pl.BlockSpec((32, _ET), col),
        out_shape=jax.ShapeDtypeStruct((32, _E), jnp.float32),
    )(attrT, ndT, nsT)


_NW = 32
_BW = _E // _NW
_SC_MESH = dict(core_axis_name="c", subcore_axis_name="s")
_SC_PARAMS = pltpu.CompilerParams(use_tc_tiling_on_sc=False)


def _wid():
    return lax.axis_index("s") * 2 + lax.axis_index("c")


def _sc_gather(table, idx):
    D = table.shape[1]

    @functools.partial(
        pl.kernel,
        out_type=jax.ShapeDtypeStruct((_E, D), table.dtype),
        mesh=plsc.VectorSubcoreMesh(**_SC_MESH),
        compiler_params=_SC_PARAMS,
        scratch_types=[
            pltpu.VMEM((_BW,), jnp.int32),
            pltpu.VMEM((_BW, D), table.dtype),
            pltpu.SemaphoreType.DMA,
        ],
    )
    def k(table_hbm, idx_hbm, out_hbm, idx_v, rows_v, sem):
        base = _wid() * _BW
        pltpu.sync_copy(idx_hbm.at[pl.ds(base, _BW)], idx_v)
        pltpu.async_copy(table_hbm.at[idx_v], rows_v, sem).wait()
        pltpu.sync_copy(rows_v, out_hbm.at[pl.ds(base, _BW)])

    return k(table, idx)


def _sc_scatter_add(v, dst):
    zeros = jnp.zeros((_N, 16), jnp.float32)
    rows_per_s = _N // 16

    @functools.partial(
        pl.kernel,
        out_type=jax.ShapeDtypeStruct((2 * _N, 16), jnp.float32),
        mesh=plsc.VectorSubcoreMesh(**_SC_MESH),
        compiler_params=_SC_PARAMS,
        scratch_types=[
            pltpu.VMEM_SHARED((_N, 16), jnp.float32),
            pltpu.VMEM((_BW, 16), jnp.float32),
            pltpu.VMEM((_BW,), jnp.int32),
        ],
    )
    def k(v_hbm, dst_hbm, z_hbm, out_hbm, shared, v_v, idx_v):
        cid = lax.axis_index("c")
        sid = lax.axis_index("s")
        srow = sid * rows_per_s
        pltpu.sync_copy(z_hbm.at[pl.ds(srow, rows_per_s)],
                        shared.at[pl.ds(srow, rows_per_s)])
        plsc.subcore_barrier()
        base = _wid() * _BW
        pltpu.sync_copy(dst_hbm.at[pl.ds(base, _BW)], idx_v)
        pltpu.sync_copy(v_hbm.at[pl.ds(base, _BW)], v_v)
        pltpu.sync_copy(v_v, shared.at[idx_v], add=True)
        plsc.subcore_barrier()
        pltpu.sync_copy(shared.at[pl.ds(srow, rows_per_s)],
                        out_hbm.at[pl.ds(cid * _N + srow, rows_per_s)])

    out = k(v, dst, zeros)
    return out[:_N], out[_N:]


def _sc_scatter_slots(spay2d, pos_flat):
    @functools.partial(
        pl.kernel,
        out_type=jax.ShapeDtypeStruct((_EOUT, 16), jnp.int32),
        mesh=plsc.VectorSubcoreMesh(**_SC_MESH),
        compiler_params=_SC_PARAMS,
        scratch_types=[
            pltpu.VMEM((_BW,), jnp.int32),
            pltpu.VMEM((_BW, 16), jnp.int32),
        ],
    )
    def k(pay_hbm, pos_hbm, out_hbm, pos_v, rows_v):
        base = _wid() * _BW
        pltpu.sync_copy(pos_hbm.at[pl.ds(base, _BW)], pos_v)
        pltpu.sync_copy(pay_hbm.at[pl.ds(base, _BW)], rows_v)
        pltpu.sync_copy(rows_v, out_hbm.at[pos_v])

    return k(spay2d, pos_flat)


def _sel_body(scat_ref, tot_ref, sel_ref):
    i = pl.program_id(0)
    rows = scat_ref.shape[0]
    p = jax.lax.broadcasted_iota(jnp.int32, (rows, 1), 0) + i * rows
    sel_ref[...] = jnp.where(p < tot_ref[0, 0], scat_ref[:, 0:1], 0)


def _sel_from_scat(scat, tot16):
    rows = _E // 16
    return pl.pallas_call(
        _sel_body,
        grid=(16,),
        in_specs=[
            pl.BlockSpec((rows, 16), lambda i: (i, 0)),
            pl.BlockSpec((1, 16), lambda i: (0, 0)),
        ],
        out_specs=pl.BlockSpec((rows, 1), lambda i: (i, 0)),
        out_shape=jax.ShapeDtypeStruct((_E, 1), jnp.int32),
    )(scat[:_E], tot16)


def kernel(X, S, RP, Seg, bid, emb_table):
    Xf = X.reshape(_N, 12)
    H = _hsin(S, RP, emb_table)
    ang, frm = _geometry(Xf.T)
    node_attr = jnp.concatenate([H, ang.T], axis=1)
    pack = jnp.concatenate([Xf, frm.T, jnp.zeros((_N, 11), jnp.float32)],
                           axis=1)
    vals, idxs, cnt = _dist_topk(X, bid, Seg)
    pos, ipay, spay, tot16 = _positions(cnt, idxs)

    dst_flat = idxs.reshape(_E)
    gath = _sc_gather(pack, dst_flat)

    gdT = gath.T
    gsT = jnp.broadcast_to(pack.T[:, :, None], (32, _N, _K)).reshape(32, _E)
    vals_row = vals.reshape(1, _E)
    attrT, vT = _edge_geom(gdT, gsT, vals_row)

    part0, part1 = _sc_scatter_add(vT.T, dst_flat)
    nvecs = _nforce_finish(part0, part1, tot16,
                           gath[0:1, :], pack[0:1, :], idxs[0:1, 0:1])

    nvd = _sc_gather(nvecs, dst_flat)
    nvdT = nvd.T
    nvsT = jnp.broadcast_to(nvecs.T[:, :, None], (16, _N, _K)).reshape(16, _E)
    payT = _assemble(attrT, nvdT, nvsT)

    gpay = payT.T
    ipay2d = ipay.reshape(_E, 16)
    spay2d = spay.reshape(_E, 16)
    scat = _sc_scatter_slots(spay2d, pos.reshape(_E))
    sel = _sel_from_scat(scat, tot16).reshape(_E)
    edge_attr = _sc_gather(gpay, sel)
    ints = _sc_gather(ipay2d, sel)
    edges = jnp.stack([ints[:, 0], ints[:, 1]], axis=0)
    return (node_attr, edges, edge_attr)
